# baseline (speedup 1.0000x reference)
import functools

import jax
import jax.numpy as jnp
from jax import lax
from jax.experimental import pallas as pl
from jax.experimental.pallas import tpu as pltpu
from jax.experimental.pallas import tpu_sc as plsc

D1, D2, D3 = 300, 64, 32
DOUT = D1 + D2 + D3
NPOS = 101
B = 16384
NC, NS = 2, 16
NW = NC * NS
BPW = B // NW
C = 128
NCHUNK = BPW // C


def _body(words_hbm, pre_hbm, lrn_hbm, pos_hbm, out_hbm,
          idx_v, pos_v, r1_v, r2_v, comb_v, sem1, sem2, sem3):
    cid = lax.axis_index("c")
    sid = lax.axis_index("s")
    wid = sid * NC + cid
    base = wid * BPW

    pltpu.sync_copy(words_hbm.at[pl.ds(base, BPW)], idx_v)
    pltpu.sync_copy(pos_hbm, pos_v)

    lanes = lax.iota(jnp.int32, 16)
    tail_mask = lanes >= 4

    for c in range(NCHUNK):
        row0 = base + c * C
        idx_c = idx_v.at[pl.ds(c * C, C)]

        cp2 = pltpu.async_copy(lrn_hbm.at[idx_c], r2_v, sem2)

        @pl.loop(0, C // 16)
        def _fetch(j):
            vec = idx_v[pl.ds(c * C + j * 16, 16)]
            for k in range(16):
                pltpu.async_copy(
                    pre_hbm.at[pl.ds(vec[k], 1)],
                    r1_v.at[pl.ds(j * 16 + k, 1)], sem1)

        pltpu.make_async_copy(pre_hbm.at[pl.ds(0, C)], r1_v, sem1).wait()
        cp2.wait()

        @pl.loop(0, C, unroll=2)
        def _asm(i):
            for off in range(0, 288, 16):
                comb_v[i, pl.ds(off, 16)] = r1_v[i, pl.ds(off, 16)]
            irow = jnp.full((16,), i, jnp.int32)
            t = plsc.load_gather(r1_v, [irow, (D1 - 16) + lanes])
            plsc.store_scatter(
                comb_v, [irow, (D1 - 16) + lanes], t, mask=tail_mask)
            for off in range(0, D2, 16):
                t2 = r2_v[i, pl.ds(off, 16)]
                plsc.store_scatter(
                    comb_v, [irow, (D1 + off) + lanes], t2)
            pidx = jnp.minimum(row0 + i, NPOS - 1)
            for off in range(0, D3, 16):
                t3 = pos_v[pidx, pl.ds(off, 16)]
                plsc.store_scatter(
                    comb_v, [irow, (D1 + D2 + off) + lanes], t3)

        pltpu.sync_copy(comb_v, out_hbm.at[pl.ds(row0, C)])


@jax.jit
def _embed(words, pretrained, learned, pos_table):
    mesh = plsc.VectorSubcoreMesh(core_axis_name="c", subcore_axis_name="s")
    f = functools.partial(
        pl.kernel,
        mesh=mesh,
        compiler_params=pltpu.CompilerParams(
            use_tc_tiling_on_sc=False, needs_layout_passes=False),
        out_type=jax.ShapeDtypeStruct((B, DOUT), jnp.float32),
        scratch_types=[
            pltpu.VMEM((BPW,), jnp.int32),
            pltpu.VMEM((NPOS, D3), jnp.float32),
            pltpu.VMEM((C, D1), jnp.float32),
            pltpu.VMEM((C, D2), jnp.float32),
            pltpu.VMEM((C, DOUT), jnp.float32),
            pltpu.SemaphoreType.DMA,
            pltpu.SemaphoreType.DMA,
            pltpu.SemaphoreType.DMA,
        ],
    )(_body)
    return f(words, pretrained, learned, pos_table)


def kernel(words, pretrained, learned, pos_table):
    return _embed(words.astype(jnp.int32), pretrained, learned, pos_table)


# speedup vs baseline: 2.4272x; 2.4272x over previous
import functools

import jax
import jax.numpy as jnp
from jax import lax
from jax.experimental import pallas as pl
from jax.experimental.pallas import tpu as pltpu
from jax.experimental.pallas import tpu_sc as plsc

D1, D2, D3 = 300, 64, 32
DOUT = D1 + D2 + D3
NPOS = 101
B = 16384
NC, NS = 2, 16
NW = NC * NS
BPW = B // NW
C = 64
NCHUNK = BPW // C
G = 16
NG = C // G


def _body(words_hbm, pre_hbm, lrn_hbm, pos_hbm, out_hbm,
          idx_v, pos_v, r1_v, r2_v, comb_v, sub_s, sem1, sem2):
    cid = lax.axis_index("c")
    sid = lax.axis_index("s")
    wid = sid * NC + cid
    base = wid * BPW

    pltpu.sync_copy(words_hbm.at[pl.ds(base, BPW)], idx_v)
    pltpu.sync_copy(pos_hbm, pos_v)

    lanes = lax.iota(jnp.int32, 16)
    tail_mask = lanes >= 4

    @pl.loop(0, NCHUNK)
    def _chunk(c):
        row0 = base + c * C

        @pl.loop(0, NG)
        def _group(g):
            vec = idx_v[pl.ds(c * C + g * G, G)]
            for k in range(G):
                w = vec[k]
                sub_s[k] = w % 8
                blk = pl.multiple_of((w // 8) * 8, 8)
                pltpu.async_copy(
                    pre_hbm.at[pl.ds(blk, 8)], r1_v.at[pl.ds(k * 8, 8)],
                    sem1)
                pltpu.async_copy(
                    lrn_hbm.at[pl.ds(blk, 8)], r2_v.at[pl.ds(k * 8, 8)],
                    sem2)
            pltpu.make_async_copy(
                pre_hbm.at[pl.ds(0, G * 8)], r1_v, sem1).wait()
            pltpu.make_async_copy(
                lrn_hbm.at[pl.ds(0, G * 8)], r2_v, sem2).wait()

            @pl.loop(0, G)
            def _asm(k):
                s = k * 8 + sub_s[k]
                i = g * G + k
                for off in range(0, 288, 16):
                    comb_v[i, pl.ds(off, 16)] = r1_v[s, pl.ds(off, 16)]
                irow = jnp.full((16,), i, jnp.int32)
                srow = jnp.full((16,), s, jnp.int32)
                t = plsc.load_gather(r1_v, [srow, (D1 - 16) + lanes])
                plsc.store_scatter(
                    comb_v, [irow, (D1 - 16) + lanes], t, mask=tail_mask)
                for off in range(0, D2, 16):
                    t2 = r2_v[s, pl.ds(off, 16)]
                    plsc.store_scatter(
                        comb_v, [irow, (D1 + off) + lanes], t2)
                pidx = jnp.minimum(row0 + i, NPOS - 1)
                prow = jnp.full((16,), pidx, jnp.int32)
                for off in range(0, D3, 16):
                    t3 = plsc.load_gather(pos_v, [prow, off + lanes])
                    plsc.store_scatter(
                        comb_v, [irow, (D1 + D2 + off) + lanes], t3)

        pltpu.sync_copy(
            comb_v, out_hbm.at[pl.ds(pl.multiple_of(row0, 8), C)])


@jax.jit
def _embed(words, pretrained, learned, pos_table):
    mesh = plsc.VectorSubcoreMesh(core_axis_name="c", subcore_axis_name="s")
    f = functools.partial(
        pl.kernel,
        mesh=mesh,
        compiler_params=pltpu.CompilerParams(needs_layout_passes=False),
        out_type=jax.ShapeDtypeStruct((B, DOUT), jnp.float32),
        scratch_types=[
            pltpu.VMEM((BPW,), jnp.int32),
            pltpu.VMEM((NPOS, D3), jnp.float32),
            pltpu.VMEM((G * 8, D1), jnp.float32),
            pltpu.VMEM((G * 8, D2), jnp.float32),
            pltpu.VMEM((C, DOUT), jnp.float32),
            pltpu.SMEM((G,), jnp.int32),
            pltpu.SemaphoreType.DMA,
            pltpu.SemaphoreType.DMA,
        ],
    )(_body)
    return f(words, pretrained, learned, pos_table)


def kernel(words, pretrained, learned, pos_table):
    return _embed(words.astype(jnp.int32), pretrained, learned, pos_table)
